# baseline (speedup 1.0000x reference)
import functools

import jax
import jax.numpy as jnp
from jax import lax
from jax.experimental import pallas as pl
from jax.experimental.pallas import tpu as pltpu
from jax.experimental.pallas import tpu_sc as plsc

BATCH = 16384
NCOLS = 5
DIM = 32

_info = plsc.get_sparse_core_info()
_NC, _NS = _info.num_cores, _info.num_subcores
_NW = _NC * _NS
_BPW = BATCH // _NW


def _emb_body(i0, i1, i2, i3, i4, w0, w1, w2, w3, w4, out,
              idx_v, rows_v, sem):
    idxs = [i0, i1, i2, i3, i4]
    tables = [w0, w1, w2, w3, w4]
    wid = lax.axis_index("s") * _NC + lax.axis_index("c")
    base = wid * _BPW
    for t in range(NCOLS):
        pltpu.sync_copy(idxs[t].at[pl.ds(base, _BPW)], idx_v[t])
    copies = []
    for t in range(NCOLS):
        copies.append(
            pltpu.async_copy(tables[t].at[idx_v[t]], rows_v[t], sem))
    for t in range(NCOLS):
        copies[t].wait()
        pltpu.sync_copy(rows_v[t],
                        out.at[pl.ds(base, _BPW), pl.ds(t * DIM, DIM)])


_emb = pl.kernel(
    _emb_body,
    mesh=plsc.VectorSubcoreMesh(core_axis_name="c", subcore_axis_name="s"),
    out_type=jax.ShapeDtypeStruct((BATCH, NCOLS * DIM), jnp.float32),
    scratch_types=[
        [pltpu.VMEM((_BPW,), jnp.int32) for _ in range(NCOLS)],
        [pltpu.VMEM((_BPW, DIM), jnp.float32) for _ in range(NCOLS)],
        pltpu.SemaphoreType.DMA,
    ],
    compiler_params=pltpu.CompilerParams(use_tc_tiling_on_sc=False),
)


def kernel(cat_tensor, W0, W1, W2, W3, W4):
    cols = [cat_tensor[:, t] for t in range(NCOLS)]
    return _emb(*cols, W0, W1, W2, W3, W4)


# speedup vs baseline: 1.0042x; 1.0042x over previous
import functools

import jax
import jax.numpy as jnp
from jax import lax
from jax.experimental import pallas as pl
from jax.experimental.pallas import tpu as pltpu
from jax.experimental.pallas import tpu_sc as plsc

BATCH = 16384
NCOLS = 5
DIM = 32

_info = plsc.get_sparse_core_info()
_NC, _NS = _info.num_cores, _info.num_subcores
_NW = _NC * _NS
_BPW = BATCH // _NW


def _emb_body(i0, i1, i2, i3, i4, w0, w1, w2, w3, w4, out,
              idx_v, rows_v, sem, out_sem):
    idxs = [i0, i1, i2, i3, i4]
    tables = [w0, w1, w2, w3, w4]
    wid = lax.axis_index("s") * _NC + lax.axis_index("c")
    base = wid * _BPW
    for t in range(NCOLS):
        pltpu.sync_copy(idxs[t].at[pl.ds(base, _BPW)], idx_v[t])
    copies = []
    for t in range(NCOLS):
        copies.append(
            pltpu.async_copy(tables[t].at[idx_v[t]], rows_v[t], sem))
    outs = []
    for t in range(NCOLS):
        copies[t].wait()
        outs.append(pltpu.async_copy(
            rows_v[t], out.at[pl.ds(base, _BPW), pl.ds(t * DIM, DIM)],
            out_sem))
    for t in range(NCOLS):
        outs[t].wait()


_emb = pl.kernel(
    _emb_body,
    mesh=plsc.VectorSubcoreMesh(core_axis_name="c", subcore_axis_name="s"),
    out_type=jax.ShapeDtypeStruct((BATCH, NCOLS * DIM), jnp.float32),
    scratch_types=[
        [pltpu.VMEM((_BPW,), jnp.int32) for _ in range(NCOLS)],
        [pltpu.VMEM((_BPW, DIM), jnp.float32) for _ in range(NCOLS)],
        pltpu.SemaphoreType.DMA,
        pltpu.SemaphoreType.DMA,
    ],
    compiler_params=pltpu.CompilerParams(use_tc_tiling_on_sc=False),
)


def kernel(cat_tensor, W0, W1, W2, W3, W4):
    cols = [cat_tensor[:, t] for t in range(NCOLS)]
    return _emb(*cols, W0, W1, W2, W3, W4)
